# Initial kernel scaffold; baseline (speedup 1.0000x reference)
import jax
import jax.numpy as jnp
from jax.experimental import pallas as pl


def kernel(x, W):
    raise NotImplementedError("write your pallas kernel here")


# speedup vs baseline: 5.5143x; 5.5143x over previous
import functools

import jax
import jax.numpy as jnp
from jax import lax
from jax.experimental import pallas as pl
from jax.experimental.pallas import tpu as pltpu
from jax.experimental.pallas import tpu_sc as plsc

NUM_EMB = 8
EMB_DIM = 4
MAX_NORM = 1.0

NC = 2
NS = 16
NW = NC * NS
L = 16

N = 16384 * 200
NPT = N // NW
CB = 2048
NBLK = NPT // CB

_mesh = plsc.VectorSubcoreMesh(core_axis_name="c", subcore_axis_name="s")


@functools.partial(
    pl.kernel,
    out_type=jax.ShapeDtypeStruct((N * EMB_DIM,), jnp.float32),
    mesh=_mesh,
    scratch_types=[
        pltpu.VMEM((NUM_EMB * EMB_DIM,), jnp.float32),
        pltpu.VMEM((CB,), jnp.int32),
        pltpu.VMEM((CB * EMB_DIM,), jnp.float32),
    ],
    compiler_params=pltpu.CompilerParams(needs_layout_passes=False),
)
def _emb_lookup(x_hbm, w_hbm, out_hbm, w_v, xbuf, obuf):
    wid = lax.axis_index("s") * NC + lax.axis_index("c")

    pltpu.sync_copy(w_hbm, w_v)
    lane = lax.iota(jnp.int32, L)
    row = lax.bitwise_and(lane, NUM_EMB - 1)
    base4 = row * EMB_DIM
    cols = []
    sumsq = jnp.zeros((L,), jnp.float32)
    for d in range(EMB_DIM):
        v = plsc.load_gather(w_v, [base4 + d])
        cols.append(v)
        sumsq = sumsq + v * v
    ibits = plsc.bitcast(sumsq, jnp.int32)
    seed = plsc.bitcast(jnp.int32(0x5F3759DF) - lax.shift_right_arithmetic(ibits, 1),
                        jnp.float32)
    half = sumsq * 0.5
    y = seed
    for _ in range(3):
        y = y * (1.5 - half * y * y)
    norm = sumsq * y
    scale = jnp.where(sumsq > MAX_NORM * MAX_NORM,
                      MAX_NORM / (norm + 1e-7), 1.0)
    m8 = lane < NUM_EMB
    for d in range(EMB_DIM):
        plsc.store_scatter(w_v, [base4 + d], cols[d] * scale, mask=m8)

    def blk_body(b, carry):
        base = wid * NPT + b * CB
        pltpu.sync_copy(x_hbm.at[pl.ds(base, CB)], xbuf)

        def inner(i, c):
            xv = xbuf[pl.ds(i * L, L)]
            src = xv * EMB_DIM
            pos = i * (L * EMB_DIM) + lane * EMB_DIM
            for d in range(EMB_DIM):
                val = plsc.load_gather(w_v, [src + d])
                plsc.store_scatter(obuf, [pos + d], val)
            return c

        lax.fori_loop(0, CB // L, inner, 0)
        pltpu.sync_copy(obuf, out_hbm.at[pl.ds(base * EMB_DIM, CB * EMB_DIM)])
        return carry

    lax.fori_loop(0, NBLK, blk_body, 0)


def kernel(x, W):
    out = _emb_lookup(x.reshape(-1), W.reshape(-1))
    return out.reshape(x.shape + (EMB_DIM,))


# speedup vs baseline: 5.9106x; 1.0719x over previous
import functools

import jax
import jax.numpy as jnp
from jax import lax
from jax.experimental import pallas as pl
from jax.experimental.pallas import tpu as pltpu
from jax.experimental.pallas import tpu_sc as plsc

NUM_EMB = 8
EMB_DIM = 4
MAX_NORM = 1.0

NC = 2
NS = 16
NW = NC * NS
L = 16

N = 16384 * 200
NPT = N // NW
CB = 6400
NBLK = NPT // CB
OB = CB * EMB_DIM

_mesh = plsc.VectorSubcoreMesh(core_axis_name="c", subcore_axis_name="s")


@functools.partial(
    pl.kernel,
    out_type=jax.ShapeDtypeStruct((N * EMB_DIM,), jnp.float32),
    mesh=_mesh,
    scratch_types=[
        pltpu.VMEM((NUM_EMB * EMB_DIM,), jnp.float32),
        pltpu.VMEM((CB,), jnp.int32),
        pltpu.VMEM((CB,), jnp.int32),
        pltpu.VMEM((OB,), jnp.float32),
        pltpu.VMEM((OB,), jnp.float32),
        pltpu.SemaphoreType.DMA,
        pltpu.SemaphoreType.DMA,
        pltpu.SemaphoreType.DMA,
        pltpu.SemaphoreType.DMA,
    ],
    compiler_params=pltpu.CompilerParams(needs_layout_passes=False),
)
def _emb_lookup(x_hbm, w_hbm, out_hbm, w_v, xb0, xb1, ob0, ob1,
                si0, si1, so0, so1):
    wid = lax.axis_index("s") * NC + lax.axis_index("c")
    tbase = wid * NPT
    xbufs = (xb0, xb1)
    obufs = (ob0, ob1)
    sin = (si0, si1)
    sout = (so0, so1)

    pltpu.async_copy(x_hbm.at[pl.ds(tbase, CB)], xb0, si0)

    pltpu.sync_copy(w_hbm, w_v)
    lane = lax.iota(jnp.int32, L)
    row = lax.bitwise_and(lane, NUM_EMB - 1)
    base4 = row * EMB_DIM
    cols = []
    sumsq = jnp.zeros((L,), jnp.float32)
    for d in range(EMB_DIM):
        v = plsc.load_gather(w_v, [base4 + d])
        cols.append(v)
        sumsq = sumsq + v * v
    ibits = plsc.bitcast(sumsq, jnp.int32)
    y = plsc.bitcast(jnp.int32(0x5F3759DF) - lax.shift_right_arithmetic(ibits, 1),
                     jnp.float32)
    half = sumsq * 0.5
    for _ in range(3):
        y = y * (1.5 - half * y * y)
    norm = sumsq * y
    scale = jnp.where(sumsq > MAX_NORM * MAX_NORM,
                      MAX_NORM / (norm + 1e-7), 1.0)
    m8 = lane < NUM_EMB
    for d in range(EMB_DIM):
        plsc.store_scatter(w_v, [base4 + d], cols[d] * scale, mask=m8)

    lane4 = lane * EMB_DIM
    sidx = [lane4 + d for d in range(EMB_DIM)]

    def superblock(sb, carry):
        for s in range(2):
            b = sb * 2 + s
            xb, ob = xbufs[s], obufs[s]
            @pl.when(b + 1 < NBLK)
            def _():
                pltpu.async_copy(
                    x_hbm.at[pl.ds(tbase + (b + 1) * CB, CB)],
                    xbufs[1 - s], sin[1 - s])

            pltpu.make_async_copy(
                x_hbm.at[pl.ds(0, CB)], xb, sin[s]).wait()
            @pl.when(b >= 2)
            def _():
                pltpu.make_async_copy(
                    ob, out_hbm.at[pl.ds(0, OB)], sout[s]).wait()

            @plsc.parallel_loop(0, CB // L, unroll=8)
            def _(i):
                xv = xb[pl.ds(i * L, L)]
                src = xv * EMB_DIM
                ow = ob.at[pl.ds(i * (L * EMB_DIM), L * EMB_DIM)]
                for d in range(EMB_DIM):
                    val = plsc.load_gather(w_v, [src + d])
                    plsc.store_scatter(ow, [sidx[d]], val)

            pltpu.async_copy(
                ob, out_hbm.at[pl.ds((tbase + b * CB) * EMB_DIM, OB)], sout[s])
        return carry

    lax.fori_loop(0, NBLK // 2, superblock, 0)

    for s in range(2):
        pltpu.make_async_copy(
            obufs[s], out_hbm.at[pl.ds(0, OB)], sout[s]).wait()


def kernel(x, W):
    out = _emb_lookup(x.reshape(-1), W.reshape(-1))
    return out.reshape(x.shape + (EMB_DIM,))


# speedup vs baseline: 8.0674x; 1.3649x over previous
import functools

import jax
import jax.numpy as jnp
from jax import lax
from jax.experimental import pallas as pl
from jax.experimental.pallas import tpu as pltpu
from jax.experimental.pallas import tpu_sc as plsc

NUM_EMB = 8
EMB_DIM = 4
MAX_NORM = 1.0

NC = 2
NS = 16
NW = NC * NS
L = 16

NROW = 16384
NCOL = 200
RPT = NROW // NW
RB = 16
NBLK = RPT // RB

_COL_OFF = [16 * g for g in range(NCOL // L)] + [NCOL - L]

_mesh = plsc.VectorSubcoreMesh(core_axis_name="c", subcore_axis_name="s")


@functools.partial(
    pl.kernel,
    out_type=jax.ShapeDtypeStruct((NROW, NCOL, EMB_DIM), jnp.float32),
    mesh=_mesh,
    scratch_types=[
        pltpu.VMEM((NUM_EMB, EMB_DIM), jnp.float32),
        pltpu.VMEM((RB, NCOL), jnp.int32),
        pltpu.VMEM((RB, NCOL), jnp.int32),
        pltpu.VMEM((RB, NCOL, EMB_DIM), jnp.float32),
        pltpu.VMEM((RB, NCOL, EMB_DIM), jnp.float32),
        pltpu.SemaphoreType.DMA,
        pltpu.SemaphoreType.DMA,
        pltpu.SemaphoreType.DMA,
        pltpu.SemaphoreType.DMA,
    ],
    compiler_params=pltpu.CompilerParams(
        needs_layout_passes=False, use_tc_tiling_on_sc=False),
)
def _emb_lookup(x_hbm, w_hbm, out_hbm, w_v, xb0, xb1, ob0, ob1,
                si0, si1, so0, so1):
    wid = lax.axis_index("s") * NC + lax.axis_index("c")
    tbase = wid * RPT
    xbufs = (xb0, xb1)
    obufs = (ob0, ob1)
    sin = (si0, si1)
    sout = (so0, so1)

    pltpu.async_copy(x_hbm.at[pl.ds(tbase, RB), :], xb0, si0)

    pltpu.sync_copy(w_hbm, w_v)
    lane = lax.iota(jnp.int32, L)
    row = lax.bitwise_and(lane, NUM_EMB - 1)
    dvecs = [jnp.full((L,), d, jnp.int32) for d in range(EMB_DIM)]
    cols = []
    sumsq = jnp.zeros((L,), jnp.float32)
    for d in range(EMB_DIM):
        v = plsc.load_gather(w_v, [row, dvecs[d]])
        cols.append(v)
        sumsq = sumsq + v * v
    ibits = plsc.bitcast(sumsq, jnp.int32)
    y = plsc.bitcast(jnp.int32(0x5F3759DF) - lax.shift_right_arithmetic(ibits, 1),
                     jnp.float32)
    half = sumsq * 0.5
    for _ in range(3):
        y = y * (1.5 - half * y * y)
    norm = sumsq * y
    scale = jnp.where(sumsq > MAX_NORM * MAX_NORM,
                      MAX_NORM / (norm + 1e-7), 1.0)
    m8 = lane < NUM_EMB
    for d in range(EMB_DIM):
        plsc.store_scatter(w_v, [row, dvecs[d]], cols[d] * scale, mask=m8)

    cvecs = [off + lane for off in _COL_OFF]

    def superblock(sb, carry):
        for s in range(2):
            b = sb * 2 + s
            xb, ob = xbufs[s], obufs[s]
            @pl.when(b + 1 < NBLK)
            def _():
                pltpu.async_copy(
                    x_hbm.at[pl.ds(tbase + (b + 1) * RB, RB), :],
                    xbufs[1 - s], sin[1 - s])

            pltpu.make_async_copy(
                x_hbm.at[pl.ds(0, RB), :], xb, sin[s]).wait()
            @pl.when(b >= 2)
            def _():
                pltpu.make_async_copy(
                    ob, out_hbm.at[pl.ds(0, RB), :, :], sout[s]).wait()

            @plsc.parallel_loop(0, RB, unroll=2)
            def _(r):
                orow = ob.at[r]
                for g, off in enumerate(_COL_OFF):
                    xv = xb[r, pl.ds(off, L)]
                    for d in range(EMB_DIM):
                        val = plsc.load_gather(w_v, [xv, dvecs[d]])
                        plsc.store_scatter(orow, [cvecs[g], dvecs[d]], val)

            pltpu.async_copy(
                ob, out_hbm.at[pl.ds(tbase + b * RB, RB), :, :], sout[s])
        return carry

    lax.fori_loop(0, NBLK // 2, superblock, 0)

    for s in range(2):
        pltpu.make_async_copy(
            obufs[s], out_hbm.at[pl.ds(0, RB), :, :], sout[s]).wait()


def kernel(x, W):
    return _emb_lookup(x, W)


# speedup vs baseline: 203.4429x; 25.2180x over previous
import functools

import jax
import jax.numpy as jnp
from jax import lax
from jax.experimental import pallas as pl
from jax.experimental.pallas import tpu as pltpu
from jax.experimental.pallas import tpu_sc as plsc

NUM_EMB = 8
EMB_DIM = 4
MAX_NORM = 1.0

NC = 2
NS = 16
NW = NC * NS
L = 16

NROW = 16384
NCOL = 200
CH = 4096
NCH = NROW // CH
OCH = CH * EMB_DIM
OROW = NROW * EMB_DIM

_mesh = plsc.VectorSubcoreMesh(core_axis_name="c", subcore_axis_name="s")


@functools.partial(
    pl.kernel,
    out_type=jax.ShapeDtypeStruct((NROW * NCOL * EMB_DIM,), jnp.float32),
    mesh=_mesh,
    scratch_types=[
        pltpu.VMEM((NUM_EMB * EMB_DIM,), jnp.float32),
        pltpu.VMEM((CH,), jnp.int32),
        pltpu.VMEM((CH,), jnp.int32),
        pltpu.VMEM((OCH,), jnp.float32),
        pltpu.VMEM((OCH,), jnp.float32),
        pltpu.SemaphoreType.DMA,
        pltpu.SemaphoreType.DMA,
        pltpu.SemaphoreType.DMA,
        pltpu.SemaphoreType.DMA,
    ],
    compiler_params=pltpu.CompilerParams(
        needs_layout_passes=False, use_tc_tiling_on_sc=False),
)
def _emb_lookup(xt_hbm, w_hbm, out_hbm, w_v, xb0, xb1, ob0, ob1,
                si0, si1, so0, so1):
    wid = lax.axis_index("s") * NC + lax.axis_index("c")
    xbufs = (xb0, xb1)
    obufs = (ob0, ob1)
    sin = (si0, si1)
    sout = (so0, so1)

    big = wid < 8
    col0 = jnp.where(big, 7 * wid, 6 * wid + 8)
    ntask = jnp.where(big, 7 * NCH, 6 * NCH)

    def task_col(t):
        return col0 + lax.shift_right_logical(t, 2)

    def task_off(t):
        return lax.bitwise_and(t, NCH - 1) * CH

    def start_in(t, s):
        pltpu.async_copy(
            xt_hbm.at[task_col(t), pl.ds(task_off(t), CH)], xbufs[s], sin[s])

    start_in(jnp.int32(0), 0)

    pltpu.sync_copy(w_hbm, w_v)
    lane = lax.iota(jnp.int32, L)
    row = lax.bitwise_and(lane, NUM_EMB - 1)
    base4 = row * EMB_DIM
    cols = []
    sumsq = jnp.zeros((L,), jnp.float32)
    for d in range(EMB_DIM):
        v = plsc.load_gather(w_v, [base4 + d])
        cols.append(v)
        sumsq = sumsq + v * v
    ibits = plsc.bitcast(sumsq, jnp.int32)
    y = plsc.bitcast(jnp.int32(0x5F3759DF) - lax.shift_right_arithmetic(ibits, 1),
                     jnp.float32)
    half = sumsq * 0.5
    for _ in range(3):
        y = y * (1.5 - half * y * y)
    norm = sumsq * y
    scale = jnp.where(sumsq > MAX_NORM * MAX_NORM,
                      MAX_NORM / (norm + 1e-7), 1.0)
    m8 = lane < NUM_EMB
    for d in range(EMB_DIM):
        plsc.store_scatter(w_v, [base4 + d], cols[d] * scale, mask=m8)

    def superblock(sb, carry):
        for s in range(2):
            t = sb * 2 + s
            xb, ob = xbufs[s], obufs[s]
            @pl.when(t + 1 < ntask)
            def _():
                start_in(t + 1, 1 - s)

            pltpu.make_async_copy(
                xt_hbm.at[0, pl.ds(0, CH)], xb, sin[s]).wait()
            @pl.when(t >= 2)
            def _():
                pltpu.make_async_copy(
                    ob, out_hbm.at[pl.ds(0, OCH)], sout[s]).wait()

            @plsc.parallel_loop(0, CH // 128, unroll=2)
            def _(blk):
                ib = blk * 128
                obase = blk * (128 * EMB_DIM)
                for v8 in range(8):
                    xv = xb[pl.ds(ib + 16 * v8, L)]
                    src = xv * EMB_DIM
                    for d in range(EMB_DIM):
                        val = plsc.load_gather(w_v, [src + d])
                        ob[pl.ds(obase + d * 128 + 16 * v8, L)] = val

            pltpu.async_copy(
                ob,
                out_hbm.at[pl.ds(task_col(t) * OROW + task_off(t) * EMB_DIM,
                                 OCH)],
                sout[s])
        return carry

    lax.fori_loop(0, lax.div(ntask, 2), superblock, 0)

    for s in range(2):
        pltpu.make_async_copy(
            obufs[s], out_hbm.at[pl.ds(0, OCH)], sout[s]).wait()


def kernel(x, W):
    xt = jnp.swapaxes(x, 0, 1)
    flat = _emb_lookup(xt, W.reshape(-1))
    out = (flat.reshape(NCOL, NROW // 128, EMB_DIM, 128)
               .transpose(1, 3, 0, 2)
               .reshape(NROW, NCOL, EMB_DIM))
    return out


# speedup vs baseline: 262.1344x; 1.2885x over previous
import functools

import jax
import jax.numpy as jnp
from jax import lax
from jax.experimental import pallas as pl
from jax.experimental.pallas import tpu as pltpu
from jax.experimental.pallas import tpu_sc as plsc

NUM_EMB = 8
EMB_DIM = 4
MAX_NORM = 1.0

NC = 2
NS = 16
NW = NC * NS
L = 16

NROW = 16384
NCOL = 200
CH = 4096
NCH = NROW // CH
OCH = CH * EMB_DIM
OROW = NROW * EMB_DIM
NBI = CH // 128

_mesh = plsc.VectorSubcoreMesh(core_axis_name="c", subcore_axis_name="s")


@functools.partial(
    pl.kernel,
    out_type=jax.ShapeDtypeStruct((NROW * NCOL * EMB_DIM,), jnp.float32),
    mesh=_mesh,
    scratch_types=[
        pltpu.VMEM((NUM_EMB * EMB_DIM,), jnp.float32),
        pltpu.VMEM((NBI, 1, 128), jnp.int32),
        pltpu.VMEM((NBI, 1, 128), jnp.int32),
        pltpu.VMEM((OCH,), jnp.float32),
        pltpu.VMEM((OCH,), jnp.float32),
        pltpu.SemaphoreType.DMA,
        pltpu.SemaphoreType.DMA,
        pltpu.SemaphoreType.DMA,
        pltpu.SemaphoreType.DMA,
    ],
    compiler_params=pltpu.CompilerParams(
        needs_layout_passes=False, use_tc_tiling_on_sc=False),
)
def _emb_lookup(xr_hbm, w_hbm, out_hbm, w_v, xb0, xb1, ob0, ob1,
                si0, si1, so0, so1):
    wid = lax.axis_index("s") * NC + lax.axis_index("c")
    xbufs = (xb0, xb1)
    obufs = (ob0, ob1)
    sin = (si0, si1)
    sout = (so0, so1)

    big = wid < 8
    col0 = jnp.where(big, 7 * wid, 6 * wid + 8)
    ntask = jnp.where(big, 7 * NCH, 6 * NCH)

    def task_col(t):
        return col0 + lax.shift_right_logical(t, 2)

    def task_off(t):
        return lax.bitwise_and(t, NCH - 1) * CH

    def start_in(t, s):
        j = task_col(t)
        pltpu.async_copy(
            xr_hbm.at[lax.shift_right_logical(j, 3),
                      pl.ds(lax.bitwise_and(t, NCH - 1) * NBI, NBI),
                      pl.ds(lax.bitwise_and(j, 7), 1), :],
            xbufs[s], sin[s])

    start_in(jnp.int32(0), 0)

    pltpu.sync_copy(w_hbm, w_v)
    lane = lax.iota(jnp.int32, L)
    row = lax.bitwise_and(lane, NUM_EMB - 1)
    base4 = row * EMB_DIM
    cols = []
    sumsq = jnp.zeros((L,), jnp.float32)
    for d in range(EMB_DIM):
        v = plsc.load_gather(w_v, [base4 + d])
        cols.append(v)
        sumsq = sumsq + v * v
    ibits = plsc.bitcast(sumsq, jnp.int32)
    y = plsc.bitcast(jnp.int32(0x5F3759DF) - lax.shift_right_arithmetic(ibits, 1),
                     jnp.float32)
    half = sumsq * 0.5
    for _ in range(3):
        y = y * (1.5 - half * y * y)
    norm = sumsq * y
    scale = jnp.where(sumsq > MAX_NORM * MAX_NORM,
                      MAX_NORM / (norm + 1e-7), 1.0)
    m8 = lane < NUM_EMB
    for d in range(EMB_DIM):
        plsc.store_scatter(w_v, [base4 + d], cols[d] * scale, mask=m8)

    def superblock(sb, carry):
        for s in range(2):
            t = sb * 2 + s
            xb, ob = xbufs[s], obufs[s]
            @pl.when(t + 1 < ntask)
            def _():
                start_in(t + 1, 1 - s)

            pltpu.make_async_copy(
                xr_hbm.at[0, pl.ds(0, NBI), pl.ds(0, 1), :], xb, sin[s]).wait()
            @pl.when(t >= 2)
            def _():
                pltpu.make_async_copy(
                    ob, out_hbm.at[pl.ds(0, OCH)], sout[s]).wait()

            @plsc.parallel_loop(0, NBI, unroll=2)
            def _(blk):
                obase = blk * (128 * EMB_DIM)
                for v8 in range(8):
                    xv = xb[blk, 0, pl.ds(16 * v8, L)]
                    src = xv * EMB_DIM
                    for d in range(EMB_DIM):
                        val = plsc.load_gather(w_v, [src + d])
                        ob[pl.ds(obase + d * 128 + 16 * v8, L)] = val

            pltpu.async_copy(
                ob,
                out_hbm.at[pl.ds(task_col(t) * OROW + task_off(t) * EMB_DIM,
                                 OCH)],
                sout[s])
        return carry

    lax.fori_loop(0, lax.div(ntask, 2), superblock, 0)

    for s in range(2):
        pltpu.make_async_copy(
            obufs[s], out_hbm.at[pl.ds(0, OCH)], sout[s]).wait()


def kernel(x, W):
    xr = (jnp.swapaxes(x, 0, 1)
             .reshape(NCOL // 8, 8, NROW // 128, 128)
             .transpose(0, 2, 1, 3))
    flat = _emb_lookup(xr, W.reshape(-1))
    out = (flat.reshape(NCOL, NROW // 128, EMB_DIM, 128)
               .transpose(1, 3, 0, 2)
               .reshape(NROW, NCOL, EMB_DIM))
    return out


# speedup vs baseline: 291.6496x; 1.1126x over previous
import functools

import jax
import jax.numpy as jnp
from jax import lax
from jax.experimental import pallas as pl
from jax.experimental.pallas import tpu as pltpu
from jax.experimental.pallas import tpu_sc as plsc

NUM_EMB = 8
EMB_DIM = 4
MAX_NORM = 1.0

NC = 2
NS = 16
NW = NC * NS
L = 16

NROW = 16384
NCOL = 200
CH = 4096
NCH = NROW // CH
OCH = CH * EMB_DIM
OROW = NROW * EMB_DIM
NBI = CH // 128
NTASK = NCOL * NCH // NW

_mesh = plsc.VectorSubcoreMesh(core_axis_name="c", subcore_axis_name="s")


@functools.partial(
    pl.kernel,
    out_type=jax.ShapeDtypeStruct((NROW * NCOL * EMB_DIM,), jnp.float32),
    mesh=_mesh,
    scratch_types=[
        pltpu.VMEM((NUM_EMB * EMB_DIM,), jnp.float32),
        pltpu.VMEM((NBI, 1, 128), jnp.int32),
        pltpu.VMEM((NBI, 1, 128), jnp.int32),
        pltpu.VMEM((OCH,), jnp.float32),
        pltpu.VMEM((OCH,), jnp.float32),
        pltpu.SemaphoreType.DMA,
        pltpu.SemaphoreType.DMA,
        pltpu.SemaphoreType.DMA,
        pltpu.SemaphoreType.DMA,
    ],
    compiler_params=pltpu.CompilerParams(
        needs_layout_passes=False, use_tc_tiling_on_sc=False),
)
def _emb_lookup(xr_hbm, w_hbm, out_hbm, w_v, xb0, xb1, ob0, ob1,
                si0, si1, so0, so1):
    wid = lax.axis_index("s") * NC + lax.axis_index("c")
    tid0 = wid * NTASK
    xbufs = (xb0, xb1)
    obufs = (ob0, ob1)
    sin = (si0, si1)
    sout = (so0, so1)

    def start_in(t, s):
        g = tid0 + t
        j = lax.shift_right_logical(g, 2)
        q = lax.bitwise_and(g, NCH - 1)
        pltpu.async_copy(
            xr_hbm.at[lax.shift_right_logical(j, 3),
                      pl.ds(q * NBI, NBI),
                      pl.ds(lax.bitwise_and(j, 7), 1), :],
            xbufs[s], sin[s])

    def start_out(t, s):
        g = tid0 + t
        j = lax.shift_right_logical(g, 2)
        q = lax.bitwise_and(g, NCH - 1)
        pltpu.async_copy(
            obufs[s], out_hbm.at[pl.ds(j * OROW + q * OCH, OCH)], sout[s])

    def wait_in(s):
        pltpu.make_async_copy(
            xr_hbm.at[0, pl.ds(0, NBI), pl.ds(0, 1), :], xbufs[s],
            sin[s]).wait()

    def wait_out(s):
        pltpu.make_async_copy(
            obufs[s], out_hbm.at[pl.ds(0, OCH)], sout[s]).wait()

    start_in(jnp.int32(0), 0)

    pltpu.sync_copy(w_hbm, w_v)
    lane = lax.iota(jnp.int32, L)
    row = lax.bitwise_and(lane, NUM_EMB - 1)
    base4 = row * EMB_DIM
    cols = []
    sumsq = jnp.zeros((L,), jnp.float32)
    for d in range(EMB_DIM):
        v = plsc.load_gather(w_v, [base4 + d])
        cols.append(v)
        sumsq = sumsq + v * v
    ibits = plsc.bitcast(sumsq, jnp.int32)
    y = plsc.bitcast(jnp.int32(0x5F3759DF) - lax.shift_right_arithmetic(ibits, 1),
                     jnp.float32)
    half = sumsq * 0.5
    for _ in range(3):
        y = y * (1.5 - half * y * y)
    norm = sumsq * y
    scale = jnp.where(sumsq > MAX_NORM * MAX_NORM,
                      MAX_NORM / (norm + 1e-7), 1.0)
    wcol = [c * scale for c in cols]

    def compute(xb, ob):
        @plsc.parallel_loop(0, NBI, unroll=2)
        def _(blk):
            obase = blk * (128 * EMB_DIM)
            for v8 in range(8):
                xv = xb[blk, 0, pl.ds(16 * v8, L)]
                for d in range(EMB_DIM):
                    val = jnp.take_along_axis(wcol[d], xv, axis=0)
                    ob[pl.ds(obase + d * 128 + 16 * v8, L)] = val

    def superblock(sb, carry):
        for s in range(2):
            t = sb * 2 + s
            @pl.when(t + 1 < NTASK)
            def _():
                start_in(t + 1, 1 - s)

            wait_in(s)
            @pl.when(t >= 2)
            def _():
                wait_out(s)

            compute(xbufs[s], obufs[s])
            start_out(t, s)
        return carry

    lax.fori_loop(0, NTASK // 2, superblock, 0)

    t_last = jnp.int32(NTASK - 1)
    wait_in(0)
    wait_out(0)
    compute(xb0, ob0)
    start_out(t_last, 0)

    for s in range(2):
        wait_out(s)


def kernel(x, W):
    xr = (jnp.swapaxes(x, 0, 1)
             .reshape(NCOL // 8, 8, NROW // 128, 128)
             .transpose(0, 2, 1, 3))
    flat = _emb_lookup(xr, W.reshape(-1))
    out = (flat.reshape(NCOL, NROW // 128, EMB_DIM, 128)
               .transpose(1, 3, 0, 2)
               .reshape(NROW, NCOL, EMB_DIM))
    return out


# speedup vs baseline: 292.2946x; 1.0022x over previous
import functools

import jax
import jax.numpy as jnp
from jax import lax
from jax.experimental import pallas as pl
from jax.experimental.pallas import tpu as pltpu
from jax.experimental.pallas import tpu_sc as plsc

NUM_EMB = 8
EMB_DIM = 4
MAX_NORM = 1.0

NC = 2
NS = 16
NW = NC * NS
L = 16

NROW = 16384
NCOL = 200
CH = 4096
NCH = NROW // CH
OCH = CH * EMB_DIM
OROW = NROW * EMB_DIM
NBI = CH // 128
NTASK = NCOL * NCH // NW

_mesh = plsc.VectorSubcoreMesh(core_axis_name="c", subcore_axis_name="s")


@functools.partial(
    pl.kernel,
    out_type=jax.ShapeDtypeStruct((NROW * NCOL * EMB_DIM,), jnp.float32),
    mesh=_mesh,
    scratch_types=[
        pltpu.VMEM((NUM_EMB * EMB_DIM,), jnp.float32),
        pltpu.VMEM((NBI, 1, 128), jnp.int32),
        pltpu.VMEM((NBI, 1, 128), jnp.int32),
        pltpu.VMEM((OCH,), jnp.float32),
        pltpu.VMEM((OCH,), jnp.float32),
        pltpu.SemaphoreType.DMA,
        pltpu.SemaphoreType.DMA,
        pltpu.SemaphoreType.DMA,
        pltpu.SemaphoreType.DMA,
    ],
    compiler_params=pltpu.CompilerParams(
        needs_layout_passes=False, use_tc_tiling_on_sc=False),
)
def _emb_lookup(xr_hbm, w_hbm, out_hbm, w_v, xb0, xb1, ob0, ob1,
                si0, si1, so0, so1):
    wid = lax.axis_index("s") * NC + lax.axis_index("c")
    tid0 = wid * NTASK
    xbufs = (xb0, xb1)
    obufs = (ob0, ob1)
    sin = (si0, si1)
    sout = (so0, so1)

    def start_in(t, s):
        g = tid0 + t
        j = lax.shift_right_logical(g, 2)
        q = lax.bitwise_and(g, NCH - 1)
        pltpu.async_copy(
            xr_hbm.at[lax.shift_right_logical(j, 3),
                      pl.ds(q * NBI, NBI),
                      pl.ds(lax.bitwise_and(j, 7), 1), :],
            xbufs[s], sin[s])

    def start_out(t, s):
        g = tid0 + t
        j = lax.shift_right_logical(g, 2)
        q = lax.bitwise_and(g, NCH - 1)
        pltpu.async_copy(
            obufs[s], out_hbm.at[pl.ds(j * OROW + q * OCH, OCH)], sout[s])

    def wait_in(s):
        pltpu.make_async_copy(
            xr_hbm.at[0, pl.ds(0, NBI), pl.ds(0, 1), :], xbufs[s],
            sin[s]).wait()

    def wait_out(s):
        pltpu.make_async_copy(
            obufs[s], out_hbm.at[pl.ds(0, OCH)], sout[s]).wait()

    start_in(jnp.int32(0), 0)

    pltpu.sync_copy(w_hbm, w_v)
    lane = lax.iota(jnp.int32, L)
    row = lax.bitwise_and(lane, NUM_EMB - 1)
    base4 = row * EMB_DIM
    cols = []
    sumsq = jnp.zeros((L,), jnp.float32)
    for d in range(EMB_DIM):
        v = plsc.load_gather(w_v, [base4 + d])
        cols.append(v)
        sumsq = sumsq + v * v
    ibits = plsc.bitcast(sumsq, jnp.int32)
    y = plsc.bitcast(jnp.int32(0x5F3759DF) - lax.shift_right_arithmetic(ibits, 1),
                     jnp.float32)
    half = sumsq * 0.5
    for _ in range(3):
        y = y * (1.5 - half * y * y)
    norm = sumsq * y
    scale = jnp.where(sumsq > MAX_NORM * MAX_NORM,
                      MAX_NORM / (norm + 1e-7), 1.0)
    wcol = [c * scale for c in cols]

    def compute(xb, ob):
        @plsc.parallel_loop(0, NBI, unroll=4)
        def _(blk):
            obase = blk * (128 * EMB_DIM)
            for v8 in range(8):
                xv = xb[blk, 0, pl.ds(16 * v8, L)]
                for d in range(EMB_DIM):
                    val = jnp.take_along_axis(wcol[d], xv, axis=0)
                    ob[pl.ds(obase + d * 128 + 16 * v8, L)] = val

    def superblock(sb, carry):
        for s in range(2):
            t = sb * 2 + s
            @pl.when(t + 1 < NTASK)
            def _():
                start_in(t + 1, 1 - s)

            wait_in(s)
            @pl.when(t >= 2)
            def _():
                wait_out(s)

            compute(xbufs[s], obufs[s])
            start_out(t, s)
        return carry

    lax.fori_loop(0, NTASK // 2, superblock, 0)

    t_last = jnp.int32(NTASK - 1)
    wait_in(0)
    wait_out(0)
    compute(xb0, ob0)
    start_out(t_last, 0)

    for s in range(2):
        wait_out(s)


def kernel(x, W):
    xr = (jnp.swapaxes(x, 0, 1)
             .reshape(NCOL // 8, 8, NROW // 128, 128)
             .transpose(0, 2, 1, 3))
    flat = _emb_lookup(xr, W.reshape(-1))
    out = (flat.reshape(NCOL, NROW // 128, EMB_DIM, 128)
               .transpose(1, 3, 0, 2)
               .reshape(NROW, NCOL, EMB_DIM))
    return out
